# Initial kernel scaffold; baseline (speedup 1.0000x reference)
import jax
import jax.numpy as jnp
from jax.experimental import pallas as pl


def kernel(x):
    raise NotImplementedError("write your pallas kernel here")


# speedup vs baseline: 9.7572x; 9.7572x over previous
import jax
import jax.numpy as jnp
import numpy as np
from jax import lax
from jax.experimental import pallas as pl

_B = 16384
_R = 128
_C = 128

_K1 = np.uint32(0)
_K2 = np.uint32(42)
_K3 = np.uint32(42 ^ 0x1BD11BDA)

_ROT_A = (13, 15, 26, 6)
_ROT_B = (17, 29, 16, 24)


def _rotl(v, r):
    return lax.shift_left(v, jnp.uint32(r)) | lax.shift_right_logical(
        v, jnp.uint32(32 - r))


def _threefry_hash(x0, x1):
    ks = (_K1, _K2, _K3)
    x0 = x0 + ks[0]
    x1 = x1 + ks[1]
    for g in range(5):
        rots = _ROT_A if g % 2 == 0 else _ROT_B
        for r in rots:
            x0 = x0 + x1
            x1 = _rotl(x1, r)
            x1 = x0 ^ x1
        x0 = x0 + ks[(g + 1) % 3]
        x1 = x1 + ks[(g + 2) % 3] + jnp.uint32(g + 1)
    return x0, x1


def _gumbel_from_flat_index(f):
    v0, v1 = _threefry_hash(jnp.zeros_like(f), f)
    bits = v0 ^ v1
    float_bits = lax.shift_right_logical(bits, jnp.uint32(9)) | jnp.uint32(
        0x3F800000)
    floats = lax.bitcast_convert_type(float_bits, jnp.float32) - jnp.float32(1.0)
    tiny = jnp.float32(jnp.finfo(jnp.float32).tiny)
    u = lax.max(tiny, floats * (jnp.float32(1.0) - tiny) + tiny)
    return -jnp.log(-jnp.log(u))


def _body(x_ref, act_ref, ent_ref, lpa_ref):
    x = x_ref[...]
    r = lax.broadcasted_iota(jnp.uint32, (_R, _C), 0)
    c = lax.broadcasted_iota(jnp.uint32, (_R, _C), 1)
    i = r * jnp.uint32(_C) + c
    g0 = _gumbel_from_flat_index(i * jnp.uint32(2))
    g1 = _gumbel_from_flat_index(i * jnp.uint32(2) + jnp.uint32(1))

    s = jax.nn.sigmoid(x)
    comp = jnp.float32(1.0) - s
    lp0 = jnp.log(comp)
    lp1 = jnp.log(s)
    ent_ref[...] = -(lp0 * comp + lp1 * s)
    take1 = (g1 + lp1) > (g0 + lp0)
    act_ref[...] = take1.astype(jnp.int32)
    lpa_ref[...] = jnp.where(take1, lp1, lp0)


def kernel(x):
    x2 = x.reshape(_R, _C)
    act, ent, lpa = pl.pallas_call(
        _body,
        out_shape=(
            jax.ShapeDtypeStruct((_R, _C), jnp.int32),
            jax.ShapeDtypeStruct((_R, _C), jnp.float32),
            jax.ShapeDtypeStruct((_R, _C), jnp.float32),
        ),
    )(x2)
    return (act.reshape(_B, 1), ent.reshape(_B, 1), lpa.reshape(_B, 1))
